# Initial kernel scaffold; baseline (speedup 1.0000x reference)
import functools

import jax
import jax.numpy as jnp
from jax import lax
from jax.experimental import pallas as pl
from jax.experimental.pallas import tpu as pltpu
from jax.experimental.pallas import tpu_sc as plsc

B, L, V, D = 16384, 64, 1000000, 8
BL = B * L
H = L * D


_NC, _NS = 2, 16
_NW = _NC * _NS
_PER_W = BL // _NW
_CH = 2048
_NCH = _PER_W // _CH


def _sc_gather(table, idx):
    mesh = plsc.VectorSubcoreMesh(core_axis_name="c", subcore_axis_name="s")

    @functools.partial(
        pl.kernel,
        mesh=mesh,
        out_type=jax.ShapeDtypeStruct((BL, D), jnp.float32),
        scratch_types=[
            pltpu.VMEM((_CH,), jnp.int32),
            pltpu.VMEM((_CH, D), jnp.float32),
            pltpu.SemaphoreType.DMA,
        ],
    )
    def gather_kernel(idx_hbm, table_hbm, out_hbm, idx_v, rows_v, sem):
        wid = lax.axis_index("s") * _NC + lax.axis_index("c")
        base = wid * _PER_W

        def body(i, carry):
            start = base + i * _CH
            pltpu.sync_copy(idx_hbm.at[pl.ds(start, _CH)], idx_v)
            pltpu.async_copy(table_hbm.at[idx_v], rows_v, sem).wait()
            pltpu.sync_copy(rows_v, out_hbm.at[pl.ds(start, _CH)])
            return carry

        lax.fori_loop(0, _NCH, body, 0)

    return gather_kernel(idx, table)


_BB = 512


def _mlp(x, emb, E, W1, b1, W2, b2, W3, b3, W4, b4, W5, b5):
    def mlp_kernel(x_ref, emb_ref, E_ref, W1_ref, b1_ref, W2_ref, b2_ref,
                   W3_ref, b3_ref, W4_ref, b4_ref, W5_ref, b5_ref, out_ref):
        m = (x_ref[...] != 0).astype(jnp.float32)
        mask = jnp.dot(m, E_ref[...],
                       preferred_element_type=jnp.float32)
        h = emb_ref[...] * mask
        h = jnp.maximum(jnp.dot(h, W1_ref[...],
                                preferred_element_type=jnp.float32)
                        + b1_ref[...], 0.0)
        h = jnp.maximum(jnp.dot(h, W2_ref[...],
                                preferred_element_type=jnp.float32)
                        + b2_ref[...], 0.0)
        h = jnp.maximum(jnp.dot(h, W3_ref[...],
                                preferred_element_type=jnp.float32)
                        + b3_ref[...], 0.0)
        h = jnp.maximum(jnp.dot(h, W4_ref[...],
                                preferred_element_type=jnp.float32)
                        + b4_ref[...], 0.0)
        out_ref[...] = (jnp.dot(h, W5_ref[...],
                                preferred_element_type=jnp.float32)
                        + b5_ref[...])

    grid = (B // _BB,)
    full = lambda shape: pl.BlockSpec(shape, lambda i: (0, 0))
    return pl.pallas_call(
        mlp_kernel,
        grid=grid,
        in_specs=[
            pl.BlockSpec((_BB, L), lambda i: (i, 0)),
            pl.BlockSpec((_BB, H), lambda i: (i, 0)),
            full((L, H)),
            full((512, 512)), full((1, 512)),
            full((512, 512)), full((1, 512)),
            full((512, 256)), full((1, 256)),
            full((256, 128)), full((1, 128)),
            full((128, 2)), full((1, 2)),
        ],
        out_specs=pl.BlockSpec((_BB, 2), lambda i: (i, 0)),
        out_shape=jax.ShapeDtypeStruct((B, 2), jnp.float32),
    )(x, emb, E, W1, b1, W2, b2, W3, b3, W4, b4, W5, b5)


def kernel(x, table, W1, b1, W2, b2, W3, b3, W4, b4, W5, b5):
    idx = x.reshape(-1)
    emb = _sc_gather(table, idx)
    emb = emb.reshape(B, H)
    E = jnp.repeat(jnp.eye(L, dtype=jnp.float32), D, axis=1)
    return _mlp(x, emb, E,
                W1, b1.reshape(1, -1), W2, b2.reshape(1, -1),
                W3, b3.reshape(1, -1), W4, b4.reshape(1, -1),
                W5, b5.reshape(1, -1))


# speedup vs baseline: 1.6765x; 1.6765x over previous
import functools

import jax
import jax.numpy as jnp
from jax import lax
from jax.experimental import pallas as pl
from jax.experimental.pallas import tpu as pltpu
from jax.experimental.pallas import tpu_sc as plsc

B, L, V, D = 16384, 64, 1000000, 8
BL = B * L
H = L * D
NSLAB = B // 8


_NC, _NS = 2, 16
_NW = _NC * _NS
_PER_W = BL // _NW
_CH = 2048
_NCH = _PER_W // _CH
_SLAB_PER_CH = _CH // 512


def _sc_gather(table, idx):
    mesh = plsc.VectorSubcoreMesh(core_axis_name="c", subcore_axis_name="s")

    @functools.partial(
        pl.kernel,
        mesh=mesh,
        compiler_params=pltpu.CompilerParams(use_tc_tiling_on_sc=False),
        out_type=jax.ShapeDtypeStruct((BL, D), jnp.float32),
        scratch_types=[
            pltpu.VMEM((_CH,), jnp.int32),
            pltpu.VMEM((_CH, D), jnp.float32),
            pltpu.SemaphoreType.DMA,
        ],
    )
    def gather_kernel(idx_hbm, table_hbm, out_hbm, idx_v, rows_v, sem):
        wid = lax.axis_index("s") * _NC + lax.axis_index("c")
        base = wid * _PER_W

        def body(i, carry):
            start = base + i * _CH
            pltpu.sync_copy(idx_hbm.at[pl.ds(start, _CH)], idx_v)
            pltpu.async_copy(table_hbm.at[idx_v], rows_v, sem).wait()
            pltpu.sync_copy(rows_v, out_hbm.at[pl.ds(start, _CH)])
            return carry

        lax.fori_loop(0, _NCH, body, 0)

    return gather_kernel(idx, table)


_BB = 512


def _mlp(x, emb4, E, W1, b1, W2, b2, W3, b3, W4, b4, W5, b5):
    def mlp_kernel(x_ref, emb_ref, E_ref, W1_ref, b1_ref, W2_ref, b2_ref,
                   W3_ref, b3_ref, W4_ref, b4_ref, W5_ref, b5_ref, out_ref):
        m = (x_ref[...] != 0).astype(jnp.float32)
        mask = jnp.dot(m, E_ref[...],
                       preferred_element_type=jnp.float32)
        h = None
        for ct in range(4):
            e = emb_ref[:, ct, :, :].reshape(_BB, 128)
            e = e * mask[:, 128 * ct:128 * (ct + 1)]
            part = jnp.dot(e, W1_ref[pl.ds(128 * ct, 128), :],
                           preferred_element_type=jnp.float32)
            h = part if h is None else h + part
        h = jnp.maximum(h + b1_ref[...], 0.0)
        h = jnp.maximum(jnp.dot(h, W2_ref[...],
                                preferred_element_type=jnp.float32)
                        + b2_ref[...], 0.0)
        h = jnp.maximum(jnp.dot(h, W3_ref[...],
                                preferred_element_type=jnp.float32)
                        + b3_ref[...], 0.0)
        h = jnp.maximum(jnp.dot(h, W4_ref[...],
                                preferred_element_type=jnp.float32)
                        + b4_ref[...], 0.0)
        out_ref[...] = (jnp.dot(h, W5_ref[...],
                                preferred_element_type=jnp.float32)
                        + b5_ref[...])

    grid = (B // _BB,)
    full = lambda shape: pl.BlockSpec(shape, lambda i: tuple(0 for _ in shape))
    return pl.pallas_call(
        mlp_kernel,
        grid=grid,
        in_specs=[
            pl.BlockSpec((_BB, L), lambda i: (i, 0)),
            pl.BlockSpec((_BB // 8, 4, 8, 128), lambda i: (i, 0, 0, 0)),
            full((L, H)),
            full((512, 512)), full((1, 512)),
            full((512, 512)), full((1, 512)),
            full((512, 256)), full((1, 256)),
            full((256, 128)), full((1, 128)),
            full((128, 2)), full((1, 2)),
        ],
        out_specs=pl.BlockSpec((_BB, 2), lambda i: (i, 0)),
        out_shape=jax.ShapeDtypeStruct((B, 2), jnp.float32),
    )(x, emb4, E, W1, b1, W2, b2, W3, b3, W4, b4, W5, b5)


def kernel(x, table, W1, b1, W2, b2, W3, b3, W4, b4, W5, b5):
    x4 = x.reshape(NSLAB, 8, 4, 16)
    idx = x4.transpose(0, 2, 1, 3).reshape(-1)
    emb4 = _sc_gather(table, idx).reshape(NSLAB, 4, 8, 128)
    E = jnp.repeat(jnp.eye(L, dtype=jnp.float32), D, axis=1)
    return _mlp(x, emb4, E,
                W1, b1.reshape(1, -1), W2, b2.reshape(1, -1),
                W3, b3.reshape(1, -1), W4, b4.reshape(1, -1),
                W5, b5.reshape(1, -1))


# speedup vs baseline: 1.6777x; 1.0007x over previous
import functools

import jax
import jax.numpy as jnp
from jax import lax
from jax.experimental import pallas as pl
from jax.experimental.pallas import tpu as pltpu
from jax.experimental.pallas import tpu_sc as plsc

B, L, V, D = 16384, 64, 1000000, 8
BL = B * L
H = L * D
NSLAB = B // 8


_NC, _NS = 2, 16
_NW = _NC * _NS
_PER_W = BL // _NW
_CH = 2048
_NCH = _PER_W // _CH
_SLAB_PER_CH = _CH // 512


def _sc_gather(table, idx):
    mesh = plsc.VectorSubcoreMesh(core_axis_name="c", subcore_axis_name="s")

    @functools.partial(
        pl.kernel,
        mesh=mesh,
        compiler_params=pltpu.CompilerParams(use_tc_tiling_on_sc=False),
        out_type=jax.ShapeDtypeStruct((BL, D), jnp.float32),
        scratch_types=[
            pltpu.VMEM((_CH,), jnp.int32),
            pltpu.VMEM((_CH, D), jnp.float32),
            pltpu.SemaphoreType.DMA,
        ],
    )
    def gather_kernel(idx_hbm, table_hbm, out_hbm, idx_v, rows_v, sem):
        wid = lax.axis_index("s") * _NC + lax.axis_index("c")
        base = wid * _PER_W

        def body(i, carry):
            start = base + i * _CH
            pltpu.sync_copy(idx_hbm.at[pl.ds(start, _CH)], idx_v)
            pltpu.async_copy(table_hbm.at[idx_v], rows_v, sem).wait()
            pltpu.sync_copy(rows_v, out_hbm.at[pl.ds(start, _CH)])
            return carry

        lax.fori_loop(0, _NCH, body, 0)

    return gather_kernel(idx, table)


_BB = 512


def _mlp(x, emb4, E, W1, b1, W2, b2, W3, b3, W4, b4, W5, b5):
    def mlp_kernel(x_ref, emb_ref, E_ref, W1_ref, b1_ref, W2_ref, b2_ref,
                   W3_ref, b3_ref, W4_ref, b4_ref, W5_ref, b5_ref, out_ref):
        m = (x_ref[...] != 0).astype(jnp.float32)
        mask = jnp.dot(m, E_ref[...],
                       preferred_element_type=jnp.float32)
        e4 = emb_ref[...].reshape(_BB // 8, 4, 8, 128)
        h = None
        for ct in range(4):
            e = e4[:, ct, :, :].reshape(_BB, 128)
            e = e * mask[:, 128 * ct:128 * (ct + 1)]
            part = jnp.dot(e, W1_ref[pl.ds(128 * ct, 128), :],
                           preferred_element_type=jnp.float32)
            h = part if h is None else h + part
        h = jnp.maximum(h + b1_ref[...], 0.0)
        h = jnp.maximum(jnp.dot(h, W2_ref[...],
                                preferred_element_type=jnp.float32)
                        + b2_ref[...], 0.0)
        h = jnp.maximum(jnp.dot(h, W3_ref[...],
                                preferred_element_type=jnp.float32)
                        + b3_ref[...], 0.0)
        h = jnp.maximum(jnp.dot(h, W4_ref[...],
                                preferred_element_type=jnp.float32)
                        + b4_ref[...], 0.0)
        out_ref[...] = (jnp.dot(h, W5_ref[...],
                                preferred_element_type=jnp.float32)
                        + b5_ref[...])

    grid = (B // _BB,)
    full = lambda shape: pl.BlockSpec(shape, lambda i: tuple(0 for _ in shape))
    return pl.pallas_call(
        mlp_kernel,
        grid=grid,
        in_specs=[
            pl.BlockSpec((_BB, L), lambda i: (i, 0)),
            pl.BlockSpec((_BB * H // 128, 128), lambda i: (i, 0)),
            full((L, H)),
            full((512, 512)), full((1, 512)),
            full((512, 512)), full((1, 512)),
            full((512, 256)), full((1, 256)),
            full((256, 128)), full((1, 128)),
            full((128, 2)), full((1, 2)),
        ],
        out_specs=pl.BlockSpec((_BB, 2), lambda i: (i, 0)),
        out_shape=jax.ShapeDtypeStruct((B, 2), jnp.float32),
    )(x, emb4, E, W1, b1, W2, b2, W3, b3, W4, b4, W5, b5)


def kernel(x, table, W1, b1, W2, b2, W3, b3, W4, b4, W5, b5):
    x4 = x.reshape(NSLAB, 8, 4, 16)
    idx = x4.transpose(0, 2, 1, 3).reshape(-1)
    emb4 = _sc_gather(table, idx).reshape(BL * D // 128, 128)
    E = jnp.repeat(jnp.eye(L, dtype=jnp.float32), D, axis=1)
    return _mlp(x, emb4, E,
                W1, b1.reshape(1, -1), W2, b2.reshape(1, -1),
                W3, b3.reshape(1, -1), W4, b4.reshape(1, -1),
                W5, b5.reshape(1, -1))
